# baseline (speedup 1.0000x reference)
import functools

import jax
import jax.numpy as jnp
from jax import lax
from jax.experimental import pallas as pl
from jax.experimental.pallas import tpu as pltpu
from jax.experimental.pallas import tpu_sc as plsc

BATCH = 16384
ACTIVE = 100

_NC = 2
_NS = 16
_NW = _NC * _NS
_ROWS_PER_W = BATCH // _NW
_CHUNKS = 4
_R = _ROWS_PER_W // _CHUNKS
_G = _R // 16
_CW = _R * ACTIVE
_UNROLL = 5


def _sc_kernel(ids_hbm, vals_hbm, w_hbm, out_hbm,
               ids0, ids1, w0, w1, v0, v1, outv, sem0, sem1):
    idsv = (ids0, ids1)
    wv = (w0, w1)
    valsv = (v0, v1)
    sems = (sem0, sem1)
    wid = lax.axis_index("s") * _NC + lax.axis_index("c")
    base = wid * _ROWS_PER_W * ACTIVE

    def stage(c, b):
        off = base + c * _CW
        pltpu.sync_copy(ids_hbm.at[pl.ds(off, _CW)], idsv[b])
        pltpu.sync_copy(vals_hbm.at[pl.ds(off, _CW)], valsv[b])
        return pltpu.async_copy(w_hbm.at[idsv[b]], wv[b], sems[b])

    gathers = [None, None]
    gathers[0] = stage(0, 0)
    for c in range(_CHUNKS):
        b = c & 1
        if c + 1 < _CHUNKS:
            gathers[(c + 1) & 1] = stage(c + 1, (c + 1) & 1)
        gathers[b].wait()
        wb, vb = wv[b], valsv[b]

        def group_body(g, _):
            goff = g * (16 * ACTIVE)

            def a_body(t, acc):
                off = goff + t * (16 * _UNROLL)
                for k in range(_UNROLL):
                    o = off + k * 16
                    acc = acc + wb[pl.ds(o, 16)] * vb[pl.ds(o, 16)]
                return acc

            acc = lax.fori_loop(0, ACTIVE // _UNROLL, a_body,
                                jnp.zeros((16,), jnp.float32))
            y = 1.0 / (1.0 + jnp.exp(-acc))
            outv[pl.ds(c * _R + g * 16, 16)] = y
            return 0

        lax.fori_loop(0, _G, group_body, 0)

    pltpu.sync_copy(outv, out_hbm.at[pl.ds(wid * _ROWS_PER_W, _ROWS_PER_W)])


@functools.partial(jax.jit, static_argnames=())
def kernel(feature_ids_batch, feature_values_batch, W):
    ids_t = (feature_ids_batch.astype(jnp.int32)
             .reshape(BATCH // 16, 16, ACTIVE).transpose(0, 2, 1).reshape(-1))
    vals_t = (feature_values_batch
              .reshape(BATCH // 16, 16, ACTIVE).transpose(0, 2, 1).reshape(-1))

    mesh = plsc.VectorSubcoreMesh(core_axis_name="c", subcore_axis_name="s")
    out = pl.kernel(
        _sc_kernel,
        mesh=mesh,
        compiler_params=pltpu.CompilerParams(needs_layout_passes=False),
        out_type=jax.ShapeDtypeStruct((BATCH,), jnp.float32),
        scratch_types=[
            pltpu.VMEM((_CW,), jnp.int32),
            pltpu.VMEM((_CW,), jnp.int32),
            pltpu.VMEM((_CW,), jnp.float32),
            pltpu.VMEM((_CW,), jnp.float32),
            pltpu.VMEM((_CW,), jnp.float32),
            pltpu.VMEM((_CW,), jnp.float32),
            pltpu.VMEM((_ROWS_PER_W,), jnp.float32),
            pltpu.SemaphoreType.DMA,
            pltpu.SemaphoreType.DMA,
        ],
    )(ids_t, vals_t, W)
    return out.reshape(BATCH, 1)


# speedup vs baseline: 1.4970x; 1.4970x over previous
import functools

import jax
import jax.numpy as jnp
from jax import lax
from jax.experimental import pallas as pl
from jax.experimental.pallas import tpu as pltpu
from jax.experimental.pallas import tpu_sc as plsc

BATCH = 16384
ACTIVE = 100

_NC = 2
_NS = 16
_NW = _NC * _NS
_ROWS_PER_W = BATCH // _NW
_CHUNKS = 4
_R = _ROWS_PER_W // _CHUNKS
_G = _R // 16
_CW = _R * ACTIVE
_UNROLL = 5


def _sc_kernel(ids_hbm, vals_hbm, w_hbm, out_hbm,
               ids0, ids1, w0, w1, v0, v1, outv, sem0, sem1):
    idsv = (ids0, ids1)
    wv = (w0, w1)
    valsv = (v0, v1)
    sems = (sem0, sem1)
    wid = lax.axis_index("s") * _NC + lax.axis_index("c")
    base = wid * _ROWS_PER_W * ACTIVE
    lane_off = lax.iota(jnp.int32, 16) * ACTIVE

    def stage(c, b):
        off = base + c * _CW
        pltpu.sync_copy(ids_hbm.at[pl.ds(off, _CW)], idsv[b])
        pltpu.sync_copy(vals_hbm.at[pl.ds(off, _CW)], valsv[b])
        return pltpu.async_copy(w_hbm.at[idsv[b]], wv[b], sems[b])

    gathers = [None, None]
    gathers[0] = stage(0, 0)
    for c in range(_CHUNKS):
        b = c & 1
        if c + 1 < _CHUNKS:
            gathers[(c + 1) & 1] = stage(c + 1, (c + 1) & 1)
        gathers[b].wait()
        wb, vb = wv[b], valsv[b]

        def group_body(g, _):
            goff = g * (16 * ACTIVE)

            def a_body(t, acc):
                a0 = goff + t * _UNROLL
                for k in range(_UNROLL):
                    idx = lane_off + (a0 + k)
                    acc = acc + (plsc.load_gather(wb, [idx])
                                 * plsc.load_gather(vb, [idx]))
                return acc

            acc = lax.fori_loop(0, ACTIVE // _UNROLL, a_body,
                                jnp.zeros((16,), jnp.float32))
            y = 1.0 / (1.0 + jnp.exp(-acc))
            outv[pl.ds(c * _R + g * 16, 16)] = y
            return 0

        lax.fori_loop(0, _G, group_body, 0)

    pltpu.sync_copy(outv, out_hbm.at[pl.ds(wid * _ROWS_PER_W, _ROWS_PER_W)])


@functools.partial(jax.jit, static_argnames=())
def kernel(feature_ids_batch, feature_values_batch, W):
    ids_flat = feature_ids_batch.reshape(-1).astype(jnp.int32)
    vals_flat = feature_values_batch.reshape(-1)

    mesh = plsc.VectorSubcoreMesh(core_axis_name="c", subcore_axis_name="s")
    out = pl.kernel(
        _sc_kernel,
        mesh=mesh,
        compiler_params=pltpu.CompilerParams(needs_layout_passes=False),
        out_type=jax.ShapeDtypeStruct((BATCH,), jnp.float32),
        scratch_types=[
            pltpu.VMEM((_CW,), jnp.int32),
            pltpu.VMEM((_CW,), jnp.int32),
            pltpu.VMEM((_CW,), jnp.float32),
            pltpu.VMEM((_CW,), jnp.float32),
            pltpu.VMEM((_CW,), jnp.float32),
            pltpu.VMEM((_CW,), jnp.float32),
            pltpu.VMEM((_ROWS_PER_W,), jnp.float32),
            pltpu.SemaphoreType.DMA,
            pltpu.SemaphoreType.DMA,
        ],
    )(ids_flat, vals_flat, W)
    return out.reshape(BATCH, 1)
